# Initial kernel scaffold; baseline (speedup 1.0000x reference)
import jax
import jax.numpy as jnp
from jax.experimental import pallas as pl


def kernel(residue_feature, atom_feature, coords, edge_attr, edge_index, params):
    raise NotImplementedError("write your pallas kernel here")


# speedup vs baseline: 70.1867x; 70.1867x over previous
import functools

import jax
import jax.numpy as jnp
from jax import lax
from jax.experimental import pallas as pl
from jax.experimental.pallas import tpu as pltpu
from jax.experimental.pallas import tpu_sc as plsc

N_NODES = 50000
N_EDGES = 800000
HID = 64
EDGE_NF = 16

NC, NS = 2, 16
NW = NC * NS

EP = 819200
PER_W = EP // NW
PER_S = EP // NS
GG = 1024
SG = 512
NGG = PER_W // GG
NSG = PER_S // SG
NSG2 = PER_W // SG

TAB_D = 128
NACC = N_NODES + 48
ZR = NACC // NS
ORA = 3128
ORB = N_NODES - 15 * ORA

_f32 = jnp.float32


def _i32(v):
    return jnp.int32(v)


def _sc_gather_body(tab, rowg, colg, out_r, out_c, idxb, gbuf, sem):
    wid = lax.axis_index("s") * _i32(NC) + lax.axis_index("c")

    def one(idx_src, out_ref):
        def grp(g, carry):
            base = wid * _i32(PER_W) + g * _i32(GG)
            pltpu.sync_copy(idx_src.at[wid, pl.ds(g * _i32(GG // 128), GG // 128)], idxb)
            for half in range(2):
                handles = [
                    pltpu.async_copy(
                        tab.at[idxb.at[_i32(half * 4 + j)]],
                        gbuf.at[pl.ds(j * 128, 128)],
                        sem,
                    )
                    for j in range(4)
                ]
                for h in handles:
                    h.wait()
                pltpu.sync_copy(gbuf, out_ref.at[pl.ds(base + _i32(half * 512), 512)])
            return carry

        lax.fori_loop(_i32(0), _i32(NGG), grp, _i32(0))

    one(rowg, out_r)
    one(colg, out_c)


def _sc_gather(tab, rowg2, colg2):
    mesh = plsc.VectorSubcoreMesh(core_axis_name="c", subcore_axis_name="s")
    return pl.kernel(
        _sc_gather_body,
        out_type=[
            jax.ShapeDtypeStruct((EP, TAB_D), _f32),
            jax.ShapeDtypeStruct((EP, TAB_D), _f32),
        ],
        mesh=mesh,
        compiler_params=pltpu.CompilerParams(use_tc_tiling_on_sc=False),
        scratch_types=[
            pltpu.VMEM((GG // 128, 128), jnp.int32),
            pltpu.VMEM((512, TAB_D), _f32),
            pltpu.SemaphoreType.DMA,
        ],
    )(tab, rowg2, colg2)


def _sc_scatter_body(em, rows2, zeros, out1, out2, idxb, ebuf, acc):
    c = lax.axis_index("c")
    s = lax.axis_index("s")
    wid = s * _i32(NC) + c

    def zero_init():
        pltpu.sync_copy(
            zeros.at[pl.ds(s * _i32(ZR), ZR)], acc.at[pl.ds(s * _i32(ZR), ZR)]
        )

    zero_init()
    plsc.subcore_barrier()

    def grp1(g, carry):
        base = s * _i32(PER_S) + g * _i32(SG)
        pltpu.sync_copy(rows2.at[s, pl.ds(g * _i32(SG // 128), SG // 128)], idxb)

        @pl.when(c == _i32(0))
        def _():
            pltpu.sync_copy(em.at[pl.ds(base, SG), pl.ds(0, 32)], ebuf)

        @pl.when(c == _i32(1))
        def _():
            pltpu.sync_copy(em.at[pl.ds(base, SG), pl.ds(32, 32)], ebuf)

        for j in range(SG // 128):
            pltpu.sync_copy(
                ebuf.at[pl.ds(j * 128, 128)], acc.at[idxb.at[_i32(j)]], add=True
            )
        return carry

    lax.fori_loop(_i32(0), _i32(NSG), grp1, _i32(0))
    plsc.subcore_barrier()

    @pl.when(s < _i32(15))
    def _():
        pltpu.sync_copy(
            acc.at[pl.ds(s * _i32(ORA), ORA)], out1.at[c, pl.ds(s * _i32(ORA), ORA)]
        )

    @pl.when(s == _i32(15))
    def _():
        pltpu.sync_copy(
            acc.at[pl.ds(_i32(15 * ORA), ORB)], out1.at[c, pl.ds(_i32(15 * ORA), ORB)]
        )

    plsc.subcore_barrier()
    zero_init()
    plsc.subcore_barrier()

    def grp2(g, carry):
        base = s * _i32(PER_S) + c * _i32(PER_W) + g * _i32(SG)
        pltpu.sync_copy(
            rows2.at[s, pl.ds(c * _i32(PER_W // 128) + g * _i32(SG // 128), SG // 128)],
            idxb,
        )
        pltpu.sync_copy(em.at[pl.ds(base, SG), pl.ds(64, 32)], ebuf)
        for j in range(SG // 128):
            pltpu.sync_copy(
                ebuf.at[pl.ds(j * 128, 128)], acc.at[idxb.at[_i32(j)]], add=True
            )
        return carry

    lax.fori_loop(_i32(0), _i32(NSG2), grp2, _i32(0))
    plsc.subcore_barrier()

    @pl.when(s < _i32(15))
    def _():
        pltpu.sync_copy(
            acc.at[pl.ds(s * _i32(ORA), ORA), pl.ds(0, 8)],
            out2.at[c, pl.ds(s * _i32(ORA), ORA)],
        )

    @pl.when(s == _i32(15))
    def _():
        pltpu.sync_copy(
            acc.at[pl.ds(_i32(15 * ORA), ORB), pl.ds(0, 8)],
            out2.at[c, pl.ds(_i32(15 * ORA), ORB)],
        )


def _sc_scatter(em, rows2, zeros):
    mesh = plsc.VectorSubcoreMesh(core_axis_name="c", subcore_axis_name="s")
    return pl.kernel(
        _sc_scatter_body,
        out_type=[
            jax.ShapeDtypeStruct((NC, N_NODES, 32), _f32),
            jax.ShapeDtypeStruct((NC, N_NODES, 8), _f32),
        ],
        mesh=mesh,
        compiler_params=pltpu.CompilerParams(use_tc_tiling_on_sc=False),
        scratch_types=[
            pltpu.VMEM((SG // 128, 128), jnp.int32),
            pltpu.VMEM((SG, 32), _f32),
            pltpu.VMEM_SHARED((NACC, 32), _f32),
        ],
    )(em, rows2, zeros)


BE = 2048
BN = 2000


def _edge_body(gr, gc, ea, w_r, w_c, w_rad, w_e, b1, w2, b2, wc1, bc1, wc2, em):
    grv = gr[...]
    gcv = gc[...]
    diff = grv[:, HID : HID + 3] - gcv[:, HID : HID + 3]
    radial = jnp.sum(diff * diff, axis=1, keepdims=True)
    e1 = (
        jnp.dot(grv, w_r[...], preferred_element_type=_f32)
        + jnp.dot(gcv, w_c[...], preferred_element_type=_f32)
        + radial * w_rad[...]
        + jnp.dot(ea[...], w_e[...], preferred_element_type=_f32)
        + b1[...]
    )
    m = jax.nn.silu(e1)
    m = jax.nn.silu(jnp.dot(m, w2[...], preferred_element_type=_f32) + b2[...])
    cg = jax.nn.silu(jnp.dot(m, wc1[...], preferred_element_type=_f32) + bc1[...])
    cg = jnp.dot(cg, wc2[...], preferred_element_type=_f32)
    t = diff * cg
    nb = t.shape[0]
    ones = jnp.ones((nb, 1), _f32)
    em[...] = jnp.concatenate([m, t, ones, jnp.zeros((nb, 60), _f32)], axis=1)


def _edge_tc(gr, gc, ea, weights):
    full = lambda w: pl.BlockSpec(w.shape, lambda i, _n=w.ndim: (i * 0,) * _n)
    return pl.pallas_call(
        _edge_body,
        grid=(EP // BE,),
        in_specs=[
            pl.BlockSpec((BE, TAB_D), lambda i: (i, i * 0)),
            pl.BlockSpec((BE, TAB_D), lambda i: (i, i * 0)),
            pl.BlockSpec((BE, EDGE_NF), lambda i: (i, i * 0)),
        ] + [full(w) for w in weights],
        out_specs=pl.BlockSpec((BE, TAB_D), lambda i: (i, i * 0)),
        out_shape=jax.ShapeDtypeStruct((EP, TAB_D), _f32),
    )(gr, gc, ea, *weights)


def _node_body(tab, nagg, ntc, w1h, w1a, b1, w2, b2, out):
    tv = tab[...]
    h = tv[:, :HID]
    xp = tv[:, HID : HID + 3]
    agg = jnp.concatenate([nagg[0], nagg[1]], axis=1)
    tc4 = ntc[0][:, :4] + ntc[1][:, :4]
    seg = tc4[:, :3]
    cnt = jnp.maximum(tc4[:, 3:4], 1.0)
    xn = xp + seg / cnt
    hn = jax.nn.silu(
        jnp.dot(h, w1h[...], preferred_element_type=_f32)
        + jnp.dot(agg, w1a[...], preferred_element_type=_f32)
        + b1[...]
    )
    hn = jnp.dot(hn, w2[...], preferred_element_type=_f32) + b2[...]
    nb = tv.shape[0]
    out[...] = jnp.concatenate(
        [h + hn, xn, jnp.zeros((nb, TAB_D - HID - 3), _f32)], axis=1
    )


def _node_tc(tab, nagg, ntc, weights):
    full = lambda w: pl.BlockSpec(w.shape, lambda i, _n=w.ndim: (i * 0,) * _n)
    return pl.pallas_call(
        _node_body,
        grid=(N_NODES // BN,),
        in_specs=[
            pl.BlockSpec((BN, TAB_D), lambda i: (i, i * 0)),
            pl.BlockSpec((2, BN, 32), lambda i: (i * 0, i, i * 0)),
            pl.BlockSpec((2, BN, 8), lambda i: (i * 0, i, i * 0)),
        ] + [full(w) for w in weights],
        out_specs=pl.BlockSpec((BN, TAB_D), lambda i: (i, i * 0)),
        out_shape=jax.ShapeDtypeStruct((N_NODES, TAB_D), _f32),
    )(tab, nagg, ntc, *weights)


def _inproj_body(res, atom, xp, wa, wb, b, out):
    h = (
        jnp.dot(res[...], wa[...], preferred_element_type=_f32)
        + jnp.dot(atom[...], wb[...], preferred_element_type=_f32)
        + b[...]
    )
    nb = h.shape[0]
    out[...] = jnp.concatenate(
        [h, xp[...][:, :3], jnp.zeros((nb, TAB_D - HID - 3), _f32)], axis=1
    )


def _inproj_tc(res, atom, xp8, wa, wb, b):
    full = lambda w: pl.BlockSpec(w.shape, lambda i, _n=w.ndim: (i * 0,) * _n)
    return pl.pallas_call(
        _inproj_body,
        grid=(N_NODES // BN,),
        in_specs=[
            pl.BlockSpec((BN, res.shape[1]), lambda i: (i, i * 0)),
            pl.BlockSpec((BN, atom.shape[1]), lambda i: (i, i * 0)),
            pl.BlockSpec((BN, 8), lambda i: (i, i * 0)),
        ] + [full(w) for w in (wa, wb, b)],
        out_specs=pl.BlockSpec((BN, TAB_D), lambda i: (i, i * 0)),
        out_shape=jax.ShapeDtypeStruct((N_NODES, TAB_D), _f32),
    )(res, atom, xp8, wa, wb, b)


def _outproj_body(tab, w, b, out):
    out[...] = (
        jnp.dot(tab[...][:, :HID], w[...], preferred_element_type=_f32) + b[...]
    )


def _outproj_tc(tab, w, b):
    full = lambda wt: pl.BlockSpec(wt.shape, lambda i, _n=wt.ndim: (i * 0,) * _n)
    return pl.pallas_call(
        _outproj_body,
        grid=(N_NODES // BN,),
        in_specs=[pl.BlockSpec((BN, TAB_D), lambda i: (i, i * 0)), full(w), full(b)],
        out_specs=pl.BlockSpec((BN, HID), lambda i: (i, i * 0)),
        out_shape=jax.ShapeDtypeStruct((N_NODES, HID), _f32),
    )(tab, w, b)


def kernel(residue_feature, atom_feature, coords, edge_attr, edge_index, params):
    f32 = _f32
    row = edge_index[0].astype(jnp.int32)
    col = edge_index[1].astype(jnp.int32)
    pad = EP - N_EDGES
    rowg2 = jnp.concatenate([row, jnp.zeros((pad,), jnp.int32)]).reshape(
        NW, PER_W // 128, 128
    )
    colg2 = jnp.concatenate([col, jnp.zeros((pad,), jnp.int32)]).reshape(
        NW, PER_W // 128, 128
    )
    rows2 = jnp.concatenate([row, jnp.full((pad,), N_NODES, jnp.int32)]).reshape(
        NS, PER_S // 128, 128
    )
    ea_pad = jnp.concatenate(
        [edge_attr.astype(f32), jnp.zeros((pad, EDGE_NF), f32)], axis=0
    )
    xp8 = jnp.concatenate([coords.astype(f32), jnp.zeros((N_NODES, 5), f32)], axis=1)
    zeros_acc = jnp.zeros((NACC, 32), f32)

    p = params
    r2 = lambda v: v.astype(f32).reshape(1, -1)
    wa = p["W_in"][: residue_feature.shape[1]].astype(f32)
    wb = p["W_in"][residue_feature.shape[1] :].astype(f32)
    tab = _inproj_tc(
        residue_feature.astype(f32), atom_feature.astype(f32), xp8, wa, wb, r2(p["b_in"])
    )

    zpad = jnp.zeros((TAB_D - HID, HID), f32)
    for l in range(2):
        we1 = p["We1_%d" % l].astype(f32)
        w_r = jnp.concatenate([we1[:HID], zpad], axis=0)
        w_c = jnp.concatenate([we1[HID : 2 * HID], zpad], axis=0)
        w_rad = we1[2 * HID : 2 * HID + 1]
        w_e = we1[2 * HID + 1 :]
        ew = [
            w_r,
            w_c,
            w_rad,
            w_e,
            r2(p["be1_%d" % l]),
            p["We2_%d" % l].astype(f32),
            r2(p["be2_%d" % l]),
            p["Wc1_%d" % l].astype(f32),
            r2(p["bc1_%d" % l]),
            p["Wc2_%d" % l].astype(f32),
        ]
        gr, gc = _sc_gather(tab, rowg2, colg2)
        em = _edge_tc(gr, gc, ea_pad, ew)
        nagg, ntc = _sc_scatter(em, rows2, zeros_acc)
        nw = [
            p["Wn1_%d" % l][:HID].astype(f32),
            p["Wn1_%d" % l][HID:].astype(f32),
            r2(p["bn1_%d" % l]),
            p["Wn2_%d" % l].astype(f32),
            r2(p["bn2_%d" % l]),
        ]
        tab = _node_tc(tab, nagg, ntc, nw)

    out = _outproj_tc(tab, p["W_out"].astype(f32), r2(p["b_out"]))
    return out.astype(jnp.float64)


# speedup vs baseline: 73.4772x; 1.0469x over previous
import functools

import jax
import jax.numpy as jnp
from jax import lax
from jax.experimental import pallas as pl
from jax.experimental.pallas import tpu as pltpu
from jax.experimental.pallas import tpu_sc as plsc

N_NODES = 50000
N_EDGES = 800000
HID = 64
EDGE_NF = 16

NC, NS = 2, 16
NW = NC * NS

EP = 819200
PER_W = EP // NW
PER_S = EP // NS
GG = 1024
SG = 512
NGG = PER_W // GG
NSG = PER_S // SG
NSG2 = PER_W // SG

TAB_D = 128
NACC = N_NODES + 48
ZR = NACC // NS
ORA = 3128
ORB = N_NODES - 15 * ORA

_f32 = jnp.float32


def _i32(v):
    return jnp.int32(v)


def _sc_gather_body(tab, rowg, colg, out_r, out_c, idxb, gbuf, sem, semw):
    wid = lax.axis_index("s") * _i32(NC) + lax.axis_index("c")

    def one(idx_src, out_ref, first):
        def grp(g, carry):
            base = wid * _i32(PER_W) + g * _i32(GG)
            pltpu.sync_copy(idx_src.at[wid, pl.ds(g * _i32(GG // 128), GG // 128)], idxb)
            for q in range(4):
                buf = gbuf.at[pl.ds((q % 2) * 256, 256)]
                if q >= 2 or not first:
                    pltpu.make_async_copy(
                        out_ref.at[pl.ds(base, 256)], buf, semw
                    ).wait()
                else:

                    @pl.when(g > _i32(0))
                    def _():
                        pltpu.make_async_copy(
                            out_ref.at[pl.ds(base, 256)], buf, semw
                        ).wait()

                handles = [
                    pltpu.async_copy(
                        tab.at[idxb.at[_i32(q * 2 + j)]],
                        buf.at[pl.ds(j * 128, 128)],
                        sem,
                    )
                    for j in range(2)
                ]
                for h in handles:
                    h.wait()
                pltpu.async_copy(
                    buf, out_ref.at[pl.ds(base + _i32(q * 256), 256)], semw
                )
            return carry

        lax.fori_loop(_i32(0), _i32(NGG), grp, _i32(0))

    one(rowg, out_r, True)
    one(colg, out_c, False)
    for q in range(2):
        pltpu.make_async_copy(
            out_c.at[pl.ds(wid * _i32(PER_W), 256)],
            gbuf.at[pl.ds(q * 256, 256)],
            semw,
        ).wait()


def _sc_gather(tab, rowg2, colg2):
    mesh = plsc.VectorSubcoreMesh(core_axis_name="c", subcore_axis_name="s")
    return pl.kernel(
        _sc_gather_body,
        out_type=[
            jax.ShapeDtypeStruct((EP, TAB_D), _f32),
            jax.ShapeDtypeStruct((EP, TAB_D), _f32),
        ],
        mesh=mesh,
        compiler_params=pltpu.CompilerParams(use_tc_tiling_on_sc=False),
        scratch_types=[
            pltpu.VMEM((GG // 128, 128), jnp.int32),
            pltpu.VMEM((512, TAB_D), _f32),
            pltpu.SemaphoreType.DMA,
            pltpu.SemaphoreType.DMA,
        ],
    )(tab, rowg2, colg2)


def _sc_scatter_body(em, rows2, zeros, out1, out2, idxb, ebuf, acc):
    c = lax.axis_index("c")
    s = lax.axis_index("s")
    wid = s * _i32(NC) + c

    def zero_init():
        pltpu.sync_copy(
            zeros.at[pl.ds(s * _i32(ZR), ZR)], acc.at[pl.ds(s * _i32(ZR), ZR)]
        )

    zero_init()
    plsc.subcore_barrier()

    def grp1(g, carry):
        base = s * _i32(PER_S) + g * _i32(SG)
        pltpu.sync_copy(rows2.at[s, pl.ds(g * _i32(SG // 128), SG // 128)], idxb)

        @pl.when(c == _i32(0))
        def _():
            pltpu.sync_copy(em.at[pl.ds(base, SG), pl.ds(0, 32)], ebuf)

        @pl.when(c == _i32(1))
        def _():
            pltpu.sync_copy(em.at[pl.ds(base, SG), pl.ds(32, 32)], ebuf)

        for j in range(SG // 128):
            pltpu.sync_copy(
                ebuf.at[pl.ds(j * 128, 128)], acc.at[idxb.at[_i32(j)]], add=True
            )
        return carry

    lax.fori_loop(_i32(0), _i32(NSG), grp1, _i32(0))
    plsc.subcore_barrier()

    @pl.when(s < _i32(15))
    def _():
        pltpu.sync_copy(
            acc.at[pl.ds(s * _i32(ORA), ORA)], out1.at[c, pl.ds(s * _i32(ORA), ORA)]
        )

    @pl.when(s == _i32(15))
    def _():
        pltpu.sync_copy(
            acc.at[pl.ds(_i32(15 * ORA), ORB)], out1.at[c, pl.ds(_i32(15 * ORA), ORB)]
        )

    plsc.subcore_barrier()
    zero_init()
    plsc.subcore_barrier()

    def grp2(g, carry):
        base = s * _i32(PER_S) + c * _i32(PER_W) + g * _i32(SG)
        pltpu.sync_copy(
            rows2.at[s, pl.ds(c * _i32(PER_W // 128) + g * _i32(SG // 128), SG // 128)],
            idxb,
        )
        pltpu.sync_copy(em.at[pl.ds(base, SG), pl.ds(64, 32)], ebuf)
        for j in range(SG // 128):
            pltpu.sync_copy(
                ebuf.at[pl.ds(j * 128, 128)], acc.at[idxb.at[_i32(j)]], add=True
            )
        return carry

    lax.fori_loop(_i32(0), _i32(NSG2), grp2, _i32(0))
    plsc.subcore_barrier()

    @pl.when(s < _i32(15))
    def _():
        pltpu.sync_copy(
            acc.at[pl.ds(s * _i32(ORA), ORA), pl.ds(0, 8)],
            out2.at[c, pl.ds(s * _i32(ORA), ORA)],
        )

    @pl.when(s == _i32(15))
    def _():
        pltpu.sync_copy(
            acc.at[pl.ds(_i32(15 * ORA), ORB), pl.ds(0, 8)],
            out2.at[c, pl.ds(_i32(15 * ORA), ORB)],
        )


def _sc_scatter(em, rows2, zeros):
    mesh = plsc.VectorSubcoreMesh(core_axis_name="c", subcore_axis_name="s")
    return pl.kernel(
        _sc_scatter_body,
        out_type=[
            jax.ShapeDtypeStruct((NC, N_NODES, 32), _f32),
            jax.ShapeDtypeStruct((NC, N_NODES, 8), _f32),
        ],
        mesh=mesh,
        compiler_params=pltpu.CompilerParams(use_tc_tiling_on_sc=False),
        scratch_types=[
            pltpu.VMEM((SG // 128, 128), jnp.int32),
            pltpu.VMEM((SG, 32), _f32),
            pltpu.VMEM_SHARED((NACC, 32), _f32),
        ],
    )(em, rows2, zeros)


BE = 2048
BN = 2000


def _edge_body(gr, gc, ea, w_r, w_c, w_rad, w_e, b1, w2, b2, wc1, bc1, wc2, em):
    grv = gr[...]
    gcv = gc[...]
    diff = grv[:, HID : HID + 3] - gcv[:, HID : HID + 3]
    radial = jnp.sum(diff * diff, axis=1, keepdims=True)
    e1 = (
        jnp.dot(grv, w_r[...], preferred_element_type=_f32)
        + jnp.dot(gcv, w_c[...], preferred_element_type=_f32)
        + radial * w_rad[...]
        + jnp.dot(ea[...], w_e[...], preferred_element_type=_f32)
        + b1[...]
    )
    m = jax.nn.silu(e1)
    m = jax.nn.silu(jnp.dot(m, w2[...], preferred_element_type=_f32) + b2[...])
    cg = jax.nn.silu(jnp.dot(m, wc1[...], preferred_element_type=_f32) + bc1[...])
    cg = jnp.dot(cg, wc2[...], preferred_element_type=_f32)
    t = diff * cg
    nb = t.shape[0]
    ones = jnp.ones((nb, 1), _f32)
    em[...] = jnp.concatenate([m, t, ones, jnp.zeros((nb, 60), _f32)], axis=1)


def _edge_tc(gr, gc, ea, weights):
    full = lambda w: pl.BlockSpec(w.shape, lambda i, _n=w.ndim: (i * 0,) * _n)
    return pl.pallas_call(
        _edge_body,
        grid=(EP // BE,),
        in_specs=[
            pl.BlockSpec((BE, TAB_D), lambda i: (i, i * 0)),
            pl.BlockSpec((BE, TAB_D), lambda i: (i, i * 0)),
            pl.BlockSpec((BE, EDGE_NF), lambda i: (i, i * 0)),
        ] + [full(w) for w in weights],
        out_specs=pl.BlockSpec((BE, TAB_D), lambda i: (i, i * 0)),
        out_shape=jax.ShapeDtypeStruct((EP, TAB_D), _f32),
    )(gr, gc, ea, *weights)


def _node_body(tab, nagg, ntc, w1h, w1a, b1, w2, b2, out):
    tv = tab[...]
    h = tv[:, :HID]
    xp = tv[:, HID : HID + 3]
    agg = jnp.concatenate([nagg[0], nagg[1]], axis=1)
    tc4 = ntc[0][:, :4] + ntc[1][:, :4]
    seg = tc4[:, :3]
    cnt = jnp.maximum(tc4[:, 3:4], 1.0)
    xn = xp + seg / cnt
    hn = jax.nn.silu(
        jnp.dot(h, w1h[...], preferred_element_type=_f32)
        + jnp.dot(agg, w1a[...], preferred_element_type=_f32)
        + b1[...]
    )
    hn = jnp.dot(hn, w2[...], preferred_element_type=_f32) + b2[...]
    nb = tv.shape[0]
    out[...] = jnp.concatenate(
        [h + hn, xn, jnp.zeros((nb, TAB_D - HID - 3), _f32)], axis=1
    )


def _node_tc(tab, nagg, ntc, weights):
    full = lambda w: pl.BlockSpec(w.shape, lambda i, _n=w.ndim: (i * 0,) * _n)
    return pl.pallas_call(
        _node_body,
        grid=(N_NODES // BN,),
        in_specs=[
            pl.BlockSpec((BN, TAB_D), lambda i: (i, i * 0)),
            pl.BlockSpec((2, BN, 32), lambda i: (i * 0, i, i * 0)),
            pl.BlockSpec((2, BN, 8), lambda i: (i * 0, i, i * 0)),
        ] + [full(w) for w in weights],
        out_specs=pl.BlockSpec((BN, TAB_D), lambda i: (i, i * 0)),
        out_shape=jax.ShapeDtypeStruct((N_NODES, TAB_D), _f32),
    )(tab, nagg, ntc, *weights)


def _inproj_body(res, atom, xp, wa, wb, b, out):
    h = (
        jnp.dot(res[...], wa[...], preferred_element_type=_f32)
        + jnp.dot(atom[...], wb[...], preferred_element_type=_f32)
        + b[...]
    )
    nb = h.shape[0]
    out[...] = jnp.concatenate(
        [h, xp[...][:, :3], jnp.zeros((nb, TAB_D - HID - 3), _f32)], axis=1
    )


def _inproj_tc(res, atom, xp8, wa, wb, b):
    full = lambda w: pl.BlockSpec(w.shape, lambda i, _n=w.ndim: (i * 0,) * _n)
    return pl.pallas_call(
        _inproj_body,
        grid=(N_NODES // BN,),
        in_specs=[
            pl.BlockSpec((BN, res.shape[1]), lambda i: (i, i * 0)),
            pl.BlockSpec((BN, atom.shape[1]), lambda i: (i, i * 0)),
            pl.BlockSpec((BN, 8), lambda i: (i, i * 0)),
        ] + [full(w) for w in (wa, wb, b)],
        out_specs=pl.BlockSpec((BN, TAB_D), lambda i: (i, i * 0)),
        out_shape=jax.ShapeDtypeStruct((N_NODES, TAB_D), _f32),
    )(res, atom, xp8, wa, wb, b)


def _outproj_body(tab, w, b, out):
    out[...] = (
        jnp.dot(tab[...][:, :HID], w[...], preferred_element_type=_f32) + b[...]
    )


def _outproj_tc(tab, w, b):
    full = lambda wt: pl.BlockSpec(wt.shape, lambda i, _n=wt.ndim: (i * 0,) * _n)
    return pl.pallas_call(
        _outproj_body,
        grid=(N_NODES // BN,),
        in_specs=[pl.BlockSpec((BN, TAB_D), lambda i: (i, i * 0)), full(w), full(b)],
        out_specs=pl.BlockSpec((BN, HID), lambda i: (i, i * 0)),
        out_shape=jax.ShapeDtypeStruct((N_NODES, HID), _f32),
    )(tab, w, b)


def kernel(residue_feature, atom_feature, coords, edge_attr, edge_index, params):
    f32 = _f32
    row = edge_index[0].astype(jnp.int32)
    col = edge_index[1].astype(jnp.int32)
    pad = EP - N_EDGES
    rowg2 = jnp.concatenate([row, jnp.zeros((pad,), jnp.int32)]).reshape(
        NW, PER_W // 128, 128
    )
    colg2 = jnp.concatenate([col, jnp.zeros((pad,), jnp.int32)]).reshape(
        NW, PER_W // 128, 128
    )
    rows2 = jnp.concatenate([row, jnp.full((pad,), N_NODES, jnp.int32)]).reshape(
        NS, PER_S // 128, 128
    )
    ea_pad = jnp.concatenate(
        [edge_attr.astype(f32), jnp.zeros((pad, EDGE_NF), f32)], axis=0
    )
    xp8 = jnp.concatenate([coords.astype(f32), jnp.zeros((N_NODES, 5), f32)], axis=1)
    zeros_acc = jnp.zeros((NACC, 32), f32)

    p = params
    r2 = lambda v: v.astype(f32).reshape(1, -1)
    wa = p["W_in"][: residue_feature.shape[1]].astype(f32)
    wb = p["W_in"][residue_feature.shape[1] :].astype(f32)
    tab = _inproj_tc(
        residue_feature.astype(f32), atom_feature.astype(f32), xp8, wa, wb, r2(p["b_in"])
    )

    zpad = jnp.zeros((TAB_D - HID, HID), f32)
    for l in range(2):
        we1 = p["We1_%d" % l].astype(f32)
        w_r = jnp.concatenate([we1[:HID], zpad], axis=0)
        w_c = jnp.concatenate([we1[HID : 2 * HID], zpad], axis=0)
        w_rad = we1[2 * HID : 2 * HID + 1]
        w_e = we1[2 * HID + 1 :]
        ew = [
            w_r,
            w_c,
            w_rad,
            w_e,
            r2(p["be1_%d" % l]),
            p["We2_%d" % l].astype(f32),
            r2(p["be2_%d" % l]),
            p["Wc1_%d" % l].astype(f32),
            r2(p["bc1_%d" % l]),
            p["Wc2_%d" % l].astype(f32),
        ]
        gr, gc = _sc_gather(tab, rowg2, colg2)
        em = _edge_tc(gr, gc, ea_pad, ew)
        nagg, ntc = _sc_scatter(em, rows2, zeros_acc)
        nw = [
            p["Wn1_%d" % l][:HID].astype(f32),
            p["Wn1_%d" % l][HID:].astype(f32),
            r2(p["bn1_%d" % l]),
            p["Wn2_%d" % l].astype(f32),
            r2(p["bn2_%d" % l]),
        ]
        tab = _node_tc(tab, nagg, ntc, nw)

    out = _outproj_tc(tab, p["W_out"].astype(f32), r2(p["b_out"]))
    return out.astype(jnp.float64)
